# baseline (speedup 1.0000x reference)
import functools

import jax
import jax.numpy as jnp
from jax import lax
from jax.experimental import pallas as pl
from jax.experimental.pallas import tpu as pltpu
from jax.experimental.pallas import tpu_sc as plsc

B = 16384
D = 32
V = 1000000
LB = 512
NBLK = (V + LB - 1) // LB
VP = NBLK * 128
NC, NS = 2, 16
NW = NC * NS
BPW = B // NW


def _pack_body(a_ref, b_ref, c_ref, d_ref, ao_ref, bo_ref, co_ref, do_ref):
    for src, dst in ((a_ref, ao_ref), (b_ref, bo_ref), (c_ref, co_ref),
                     (d_ref, do_ref)):
        x = src[...]
        dst[...] = jnp.concatenate(
            [x[:, q * 128:(q + 1) * 128].T for q in range(4)], axis=1)


def _pack_tables(tTs):
    in_spec = pl.BlockSpec((D, LB), lambda i: (0, i))
    out_spec = pl.BlockSpec((128, 128), lambda i: (i, 0))
    return pl.pallas_call(
        _pack_body,
        grid=(NBLK,),
        in_specs=[in_spec] * 4,
        out_specs=[out_spec] * 4,
        out_shape=[jax.ShapeDtypeStruct((VP, 128), jnp.float32)] * 4,
    )(*tTs)


@functools.cache
def _make_sc_gather():
    mesh = plsc.VectorSubcoreMesh(core_axis_name="c", subcore_axis_name="s")

    @functools.partial(
        pl.kernel,
        out_type=[jax.ShapeDtypeStruct((B, 128), jnp.float32)] * 4,
        mesh=mesh,
        scratch_types=[
            pltpu.VMEM((BPW,), jnp.int32),
            pltpu.VMEM((BPW,), jnp.int32),
            pltpu.VMEM((BPW // 2, 128), jnp.float32),
            pltpu.VMEM((BPW // 2, 128), jnp.float32),
            pltpu.SemaphoreType.DMA,
            pltpu.SemaphoreType.DMA,
            pltpu.SemaphoreType.DMA,
        ],
    )
    def sc_gather(uidx_hbm, iidx_hbm, gu_hbm, gi_hbm, mu_hbm, mi_hbm,
                  gu_out, gi_out, mu_out, mi_out,
                  ju_v, ji_v, rows_a, rows_b, sema, semb, semo):
        wid = lax.axis_index("s") * NC + lax.axis_index("c")
        base = wid * BPW
        pltpu.sync_copy(uidx_hbm.at[pl.ds(base, BPW)], ju_v)
        pltpu.sync_copy(iidx_hbm.at[pl.ds(base, BPW)], ji_v)

        def to_packed(k, _):
            s = pl.ds(k * 16, 16)
            ju = ju_v[s]
            ji = ji_v[s]
            ju_v[s] = ((ju >> 9) << 7) | (ju & 127)
            ji_v[s] = ((ji >> 9) << 7) | (ji & 127)
            return ()

        lax.fori_loop(0, BPW // 16, to_packed, (), unroll=4)

        CH = BPW // 2
        bufs = (rows_a, rows_b)
        sems = (sema, semb)
        rounds = []
        for tbl, jv, out in ((gu_hbm, ju_v, gu_out), (gi_hbm, ji_v, gi_out),
                             (mu_hbm, ju_v, mu_out), (mi_hbm, ji_v, mi_out)):
            for c in range(2):
                rounds.append((tbl, jv, out, c))
        n = len(rounds)
        gathers = [None] * n
        stores = [None] * n

        def finish(i):
            _, _, out, c = rounds[i]
            gathers[i].wait()
            stores[i] = pltpu.async_copy(
                bufs[i % 2], out.at[pl.ds(base + c * CH, CH)], semo)

        for i, (tbl, jv, out, c) in enumerate(rounds):
            if i >= 2:
                stores[i - 2].wait()
            gathers[i] = pltpu.async_copy(
                tbl.at[jv.at[pl.ds(c * CH, CH)]], bufs[i % 2], sems[i % 2])
            if i >= 1:
                finish(i - 1)
        finish(n - 1)
        stores[n - 2].wait()
        stores[n - 1].wait()

    return sc_gather


BLK = 2048


def _mlp_body(uidx_ref, iidx_ref, gu_ref, gi_ref, mu_ref, mi_ref, w1_ref,
              b1_ref, w2_ref, b2_ref, w3_ref, b3_ref, wpg_ref, wph_ref,
              bp_ref, out_ref):
    uq = (uidx_ref[...] >> 7) & 3
    iq = (iidx_ref[...] >> 7) & 3

    def pick(raw, q):
        acc = jnp.zeros((BLK, D), jnp.float32)
        for c in range(4):
            sel = jnp.broadcast_to(q == c, (BLK, D))
            acc = acc + jnp.where(sel, raw[:, c * D:(c + 1) * D], 0.0)
        return acc

    gmf = pick(gu_ref[...], uq) * pick(gi_ref[...], iq)
    x = jnp.concatenate([pick(mu_ref[...], uq), pick(mi_ref[...], iq)],
                        axis=1)
    h = jnp.maximum(jnp.dot(x, w1_ref[...],
                            preferred_element_type=jnp.float32) + b1_ref[...],
                    0.0)
    h = jnp.maximum(jnp.dot(h, w2_ref[...],
                            preferred_element_type=jnp.float32) + b2_ref[...],
                    0.0)
    h = jnp.maximum(jnp.dot(h, w3_ref[...],
                            preferred_element_type=jnp.float32) + b3_ref[...],
                    0.0)
    logit = (jnp.dot(gmf, wpg_ref[...], preferred_element_type=jnp.float32)
             + jnp.dot(h, wph_ref[...], preferred_element_type=jnp.float32)
             + bp_ref[0, 0])
    out_ref[...] = jax.nn.sigmoid(logit)


def _run_mlp(uidx, iidx, gu, gi, mu, mi, W1, b1, W2, b2, W3, b3, Wpg, Wph,
             bp):
    grid = (B // BLK,)
    raw_spec = pl.BlockSpec((BLK, 128), lambda i: (i, 0))
    idx_spec = pl.BlockSpec((BLK, 1), lambda i: (i, 0))

    def whole(shape):
        return pl.BlockSpec(shape, lambda i: (0,) * len(shape))

    out = pl.pallas_call(
        _mlp_body,
        grid=grid,
        in_specs=[
            idx_spec, idx_spec,
            raw_spec, raw_spec, raw_spec, raw_spec,
            whole((64, 32)), whole((1, 32)),
            whole((32, 16)), whole((1, 16)),
            whole((16, 8)), whole((1, 8)),
            whole((32, 1)), whole((8, 1)), whole((1, 1)),
        ],
        out_specs=pl.BlockSpec((BLK, 1), lambda i: (i, 0)),
        out_shape=jax.ShapeDtypeStruct((B, 1), jnp.float32),
    )(uidx.reshape(B, 1), iidx.reshape(B, 1), gu, gi, mu, mi,
      W1, b1.reshape(1, 32), W2,
      b2.reshape(1, 16), W3, b3.reshape(1, 8), Wpg, Wph, bp.reshape(1, 1))
    return out.reshape(B)


def kernel(user_idx, item_idx, gmf_user, gmf_item, mlp_user, mlp_item,
           W1, b1, W2, b2, W3, b3, Wp, bp):
    uidx = user_idx.astype(jnp.int32)
    iidx = item_idx.astype(jnp.int32)
    packed = _pack_tables((gmf_user.T, gmf_item.T, mlp_user.T, mlp_item.T))
    gu, gi, mu, mi = _make_sc_gather()(uidx, iidx, *packed)
    Wpg = Wp[:D]
    Wph = Wp[D:]
    return _run_mlp(uidx, iidx, gu, gi, mu, mi, W1, b1, W2, b2, W3, b3,
                    Wpg, Wph, bp)


# speedup vs baseline: 2.1386x; 2.1386x over previous
import functools

import jax
import jax.numpy as jnp
from jax import lax
from jax.experimental import pallas as pl
from jax.experimental.pallas import tpu as pltpu
from jax.experimental.pallas import tpu_sc as plsc

B = 16384
D = 32
V = 1000000
LB = 2048
NBLK = (V + LB - 1) // LB
QR = LB // 4
VP = NBLK * QR
NC, NS = 2, 16
NW = NC * NS
BPW = B // NW


def _pack_body(a_ref, b_ref, c_ref, d_ref, ao_ref, bo_ref, co_ref, do_ref):
    rows = lax.broadcasted_iota(jnp.int32, (D, 128), 0)
    cols = lax.broadcasted_iota(jnp.int32, (D, 128), 1)
    eqs = [(cols == rows + D * q).astype(jnp.float32) for q in range(4)]
    cdim = (((0,), (0,)), ((), ()))
    limit = V - pl.program_id(0) * LB
    lane = lax.broadcasted_iota(jnp.int32, (D, LB), 1)
    ok = lane < limit
    for src, dst in ((a_ref, ao_ref), (b_ref, bo_ref), (c_ref, co_ref),
                     (d_ref, do_ref)):
        x = jnp.where(ok, src[...], 0.0)
        acc = lax.dot_general(x[:, 0:QR], eqs[0], cdim,
                              preferred_element_type=jnp.float32)
        for q in range(1, 4):
            acc = acc + lax.dot_general(
                x[:, q * QR:(q + 1) * QR], eqs[q], cdim,
                preferred_element_type=jnp.float32)
        dst[...] = acc


def _pack_tables(tTs):
    in_spec = pl.BlockSpec((D, LB), lambda i: (0, i))
    out_spec = pl.BlockSpec((QR, 128), lambda i: (i, 0))
    return pl.pallas_call(
        _pack_body,
        grid=(NBLK,),
        in_specs=[in_spec] * 4,
        out_specs=[out_spec] * 4,
        out_shape=[jax.ShapeDtypeStruct((VP, 128), jnp.float32)] * 4,
        compiler_params=pltpu.CompilerParams(
            fuse_transposed_lhs_in_matmul=True),
    )(*tTs)


@functools.cache
def _make_sc_gather():
    mesh = plsc.VectorSubcoreMesh(core_axis_name="c", subcore_axis_name="s")

    @functools.partial(
        pl.kernel,
        out_type=[jax.ShapeDtypeStruct((B, 128), jnp.float32)] * 4,
        mesh=mesh,
        scratch_types=[
            pltpu.VMEM((BPW,), jnp.int32),
            pltpu.VMEM((BPW,), jnp.int32),
            pltpu.VMEM((BPW // 2, 128), jnp.float32),
            pltpu.VMEM((BPW // 2, 128), jnp.float32),
            pltpu.SemaphoreType.DMA,
            pltpu.SemaphoreType.DMA,
            pltpu.SemaphoreType.DMA,
        ],
    )
    def sc_gather(uidx_hbm, iidx_hbm, gu_hbm, gi_hbm, mu_hbm, mi_hbm,
                  gu_out, gi_out, mu_out, mi_out,
                  ju_v, ji_v, rows_a, rows_b, sema, semb, semo):
        wid = lax.axis_index("s") * NC + lax.axis_index("c")
        base = wid * BPW
        pltpu.sync_copy(uidx_hbm.at[pl.ds(base, BPW)], ju_v)
        pltpu.sync_copy(iidx_hbm.at[pl.ds(base, BPW)], ji_v)

        def to_packed(k, _):
            s = pl.ds(k * 16, 16)
            ju = ju_v[s]
            ji = ji_v[s]
            ju_v[s] = ((ju >> 11) << 9) | (ju & (QR - 1))
            ji_v[s] = ((ji >> 11) << 9) | (ji & (QR - 1))
            return ()

        lax.fori_loop(0, BPW // 16, to_packed, (), unroll=4)

        CH = BPW // 2
        bufs = (rows_a, rows_b)
        sems = (sema, semb)
        rounds = []
        for tbl, jv, out in ((gu_hbm, ju_v, gu_out), (gi_hbm, ji_v, gi_out),
                             (mu_hbm, ju_v, mu_out), (mi_hbm, ji_v, mi_out)):
            for c in range(2):
                rounds.append((tbl, jv, out, c))
        n = len(rounds)
        gathers = [None] * n
        stores = [None] * n

        def finish(i):
            _, _, out, c = rounds[i]
            gathers[i].wait()
            stores[i] = pltpu.async_copy(
                bufs[i % 2], out.at[pl.ds(base + c * CH, CH)], semo)

        for i, (tbl, jv, out, c) in enumerate(rounds):
            if i >= 2:
                stores[i - 2].wait()
            gathers[i] = pltpu.async_copy(
                tbl.at[jv.at[pl.ds(c * CH, CH)]], bufs[i % 2], sems[i % 2])
            if i >= 1:
                finish(i - 1)
        finish(n - 1)
        stores[n - 2].wait()
        stores[n - 1].wait()

    return sc_gather


BLK = 2048


def _mlp_body(uidx_ref, iidx_ref, gu_ref, gi_ref, mu_ref, mi_ref, w1_ref,
              b1_ref, w2_ref, b2_ref, w3_ref, b3_ref, wpg_ref, wph_ref,
              bp_ref, out_ref):
    uq = (uidx_ref[...] >> 9) & 3
    iq = (iidx_ref[...] >> 9) & 3

    def pick(raw, q):
        acc = jnp.zeros((BLK, D), jnp.float32)
        for c in range(4):
            sel = jnp.broadcast_to(q == c, (BLK, D))
            acc = acc + jnp.where(sel, raw[:, c * D:(c + 1) * D], 0.0)
        return acc

    gmf = pick(gu_ref[...], uq) * pick(gi_ref[...], iq)
    x = jnp.concatenate([pick(mu_ref[...], uq), pick(mi_ref[...], iq)],
                        axis=1)
    h = jnp.maximum(jnp.dot(x, w1_ref[...],
                            preferred_element_type=jnp.float32) + b1_ref[...],
                    0.0)
    h = jnp.maximum(jnp.dot(h, w2_ref[...],
                            preferred_element_type=jnp.float32) + b2_ref[...],
                    0.0)
    h = jnp.maximum(jnp.dot(h, w3_ref[...],
                            preferred_element_type=jnp.float32) + b3_ref[...],
                    0.0)
    logit = (jnp.dot(gmf, wpg_ref[...], preferred_element_type=jnp.float32)
             + jnp.dot(h, wph_ref[...], preferred_element_type=jnp.float32)
             + bp_ref[0, 0])
    out_ref[...] = jax.nn.sigmoid(logit)


def _run_mlp(uidx, iidx, gu, gi, mu, mi, W1, b1, W2, b2, W3, b3, Wpg, Wph,
             bp):
    grid = (B // BLK,)
    raw_spec = pl.BlockSpec((BLK, 128), lambda i: (i, 0))
    idx_spec = pl.BlockSpec((BLK, 1), lambda i: (i, 0))

    def whole(shape):
        return pl.BlockSpec(shape, lambda i: (0,) * len(shape))

    out = pl.pallas_call(
        _mlp_body,
        grid=grid,
        in_specs=[
            idx_spec, idx_spec,
            raw_spec, raw_spec, raw_spec, raw_spec,
            whole((64, 32)), whole((1, 32)),
            whole((32, 16)), whole((1, 16)),
            whole((16, 8)), whole((1, 8)),
            whole((32, 1)), whole((8, 1)), whole((1, 1)),
        ],
        out_specs=pl.BlockSpec((BLK, 1), lambda i: (i, 0)),
        out_shape=jax.ShapeDtypeStruct((B, 1), jnp.float32),
    )(uidx.reshape(B, 1), iidx.reshape(B, 1), gu, gi, mu, mi,
      W1, b1.reshape(1, 32), W2,
      b2.reshape(1, 16), W3, b3.reshape(1, 8), Wpg, Wph, bp.reshape(1, 1))
    return out.reshape(B)


def kernel(user_idx, item_idx, gmf_user, gmf_item, mlp_user, mlp_item,
           W1, b1, W2, b2, W3, b3, Wp, bp):
    uidx = user_idx.astype(jnp.int32)
    iidx = item_idx.astype(jnp.int32)
    packed = _pack_tables((gmf_user.T, gmf_item.T, mlp_user.T, mlp_item.T))
    gu, gi, mu, mi = _make_sc_gather()(uidx, iidx, *packed)
    Wpg = Wp[:D]
    Wph = Wp[D:]
    return _run_mlp(uidx, iidx, gu, gi, mu, mi, W1, b1, W2, b2, W3, b3,
                    Wpg, Wph, bp)


# speedup vs baseline: 2.3786x; 1.1122x over previous
import functools

import jax
import jax.numpy as jnp
from jax import lax
from jax.experimental import pallas as pl
from jax.experimental.pallas import tpu as pltpu
from jax.experimental.pallas import tpu_sc as plsc

B = 16384
D = 32
V = 1000000
LB = 2048
NBLK = (V + LB - 1) // LB
QR = LB // 4
VP = NBLK * QR
NC, NS = 2, 16
NW = NC * NS
BPW = B // NW


def _pack_body(a_ref, b_ref, c_ref, d_ref, ao_ref, bo_ref, co_ref, do_ref):
    rows = lax.broadcasted_iota(jnp.int32, (128, D), 0)
    cols = lax.broadcasted_iota(jnp.int32, (128, D), 1)
    eqs = [(rows == cols + D * q).astype(jnp.float32) for q in range(4)]
    cdim = (((1,), (0,)), ((), ()))
    limit = V - pl.program_id(0) * LB
    lane = lax.broadcasted_iota(jnp.int32, (D, LB), 1)
    ok = lane < limit
    for src, dst in ((a_ref, ao_ref), (b_ref, bo_ref), (c_ref, co_ref),
                     (d_ref, do_ref)):
        x = jnp.where(ok, src[...], 0.0)
        acc = lax.dot_general(eqs[0], x[:, 0:QR], cdim,
                              preferred_element_type=jnp.float32)
        for q in range(1, 4):
            acc = acc + lax.dot_general(
                eqs[q], x[:, q * QR:(q + 1) * QR], cdim,
                preferred_element_type=jnp.float32)
        dst[...] = acc.T


def _pack_tables(tTs):
    in_spec = pl.BlockSpec((D, LB), lambda i: (0, i))
    out_spec = pl.BlockSpec((QR, 128), lambda i: (i, 0))
    return pl.pallas_call(
        _pack_body,
        grid=(NBLK,),
        in_specs=[in_spec] * 4,
        out_specs=[out_spec] * 4,
        out_shape=[jax.ShapeDtypeStruct((VP, 128), jnp.float32)] * 4,
        compiler_params=pltpu.CompilerParams(
            fuse_transposed_lhs_in_matmul=True),
    )(*tTs)


@functools.cache
def _make_sc_gather():
    mesh = plsc.VectorSubcoreMesh(core_axis_name="c", subcore_axis_name="s")

    @functools.partial(
        pl.kernel,
        out_type=[jax.ShapeDtypeStruct((B, 128), jnp.float32)] * 4,
        mesh=mesh,
        scratch_types=[
            pltpu.VMEM((BPW,), jnp.int32),
            pltpu.VMEM((BPW,), jnp.int32),
            pltpu.VMEM((BPW // 2, 128), jnp.float32),
            pltpu.VMEM((BPW // 2, 128), jnp.float32),
            pltpu.SemaphoreType.DMA,
            pltpu.SemaphoreType.DMA,
            pltpu.SemaphoreType.DMA,
        ],
    )
    def sc_gather(uidx_hbm, iidx_hbm, gu_hbm, gi_hbm, mu_hbm, mi_hbm,
                  gu_out, gi_out, mu_out, mi_out,
                  ju_v, ji_v, rows_a, rows_b, sema, semb, semo):
        wid = lax.axis_index("s") * NC + lax.axis_index("c")
        base = wid * BPW
        pltpu.sync_copy(uidx_hbm.at[pl.ds(base, BPW)], ju_v)
        pltpu.sync_copy(iidx_hbm.at[pl.ds(base, BPW)], ji_v)

        def to_packed(k, _):
            s = pl.ds(k * 16, 16)
            ju = ju_v[s]
            ji = ji_v[s]
            ju_v[s] = ((ju >> 11) << 9) | (ju & (QR - 1))
            ji_v[s] = ((ji >> 11) << 9) | (ji & (QR - 1))
            return ()

        lax.fori_loop(0, BPW // 16, to_packed, (), unroll=4)

        CH = BPW // 2
        bufs = (rows_a, rows_b)
        sems = (sema, semb)
        rounds = []
        for tbl, jv, out in ((gu_hbm, ju_v, gu_out), (gi_hbm, ji_v, gi_out),
                             (mu_hbm, ju_v, mu_out), (mi_hbm, ji_v, mi_out)):
            for c in range(2):
                rounds.append((tbl, jv, out, c))
        n = len(rounds)
        gathers = [None] * n
        stores = [None] * n

        def finish(i):
            _, _, out, c = rounds[i]
            gathers[i].wait()
            stores[i] = pltpu.async_copy(
                bufs[i % 2], out.at[pl.ds(base + c * CH, CH)], semo)

        for i, (tbl, jv, out, c) in enumerate(rounds):
            if i >= 2:
                stores[i - 2].wait()
            gathers[i] = pltpu.async_copy(
                tbl.at[jv.at[pl.ds(c * CH, CH)]], bufs[i % 2], sems[i % 2])
            if i >= 1:
                finish(i - 1)
        finish(n - 1)
        stores[n - 2].wait()
        stores[n - 1].wait()

    return sc_gather


BLK = 2048


def _mlp_body(uidx_ref, iidx_ref, gu_ref, gi_ref, mu_ref, mi_ref, w1_ref,
              b1_ref, w2_ref, b2_ref, w3_ref, b3_ref, wpg_ref, wph_ref,
              bp_ref, out_ref):
    uq = (uidx_ref[...] >> 9) & 3
    iq = (iidx_ref[...] >> 9) & 3

    def pick(raw, q):
        acc = jnp.zeros((BLK, D), jnp.float32)
        for c in range(4):
            sel = jnp.broadcast_to(q == c, (BLK, D))
            acc = acc + jnp.where(sel, raw[:, c * D:(c + 1) * D], 0.0)
        return acc

    gmf = pick(gu_ref[...], uq) * pick(gi_ref[...], iq)
    x = jnp.concatenate([pick(mu_ref[...], uq), pick(mi_ref[...], iq)],
                        axis=1)
    h = jnp.maximum(jnp.dot(x, w1_ref[...],
                            preferred_element_type=jnp.float32) + b1_ref[...],
                    0.0)
    h = jnp.maximum(jnp.dot(h, w2_ref[...],
                            preferred_element_type=jnp.float32) + b2_ref[...],
                    0.0)
    h = jnp.maximum(jnp.dot(h, w3_ref[...],
                            preferred_element_type=jnp.float32) + b3_ref[...],
                    0.0)
    logit = (jnp.dot(gmf, wpg_ref[...], preferred_element_type=jnp.float32)
             + jnp.dot(h, wph_ref[...], preferred_element_type=jnp.float32)
             + bp_ref[0, 0])
    out_ref[...] = jax.nn.sigmoid(logit)


def _run_mlp(uidx, iidx, gu, gi, mu, mi, W1, b1, W2, b2, W3, b3, Wpg, Wph,
             bp):
    grid = (B // BLK,)
    raw_spec = pl.BlockSpec((BLK, 128), lambda i: (i, 0))
    idx_spec = pl.BlockSpec((BLK, 1), lambda i: (i, 0))

    def whole(shape):
        return pl.BlockSpec(shape, lambda i: (0,) * len(shape))

    out = pl.pallas_call(
        _mlp_body,
        grid=grid,
        in_specs=[
            idx_spec, idx_spec,
            raw_spec, raw_spec, raw_spec, raw_spec,
            whole((64, 32)), whole((1, 32)),
            whole((32, 16)), whole((1, 16)),
            whole((16, 8)), whole((1, 8)),
            whole((32, 1)), whole((8, 1)), whole((1, 1)),
        ],
        out_specs=pl.BlockSpec((BLK, 1), lambda i: (i, 0)),
        out_shape=jax.ShapeDtypeStruct((B, 1), jnp.float32),
    )(uidx.reshape(B, 1), iidx.reshape(B, 1), gu, gi, mu, mi,
      W1, b1.reshape(1, 32), W2,
      b2.reshape(1, 16), W3, b3.reshape(1, 8), Wpg, Wph, bp.reshape(1, 1))
    return out.reshape(B)


def kernel(user_idx, item_idx, gmf_user, gmf_item, mlp_user, mlp_item,
           W1, b1, W2, b2, W3, b3, Wp, bp):
    uidx = user_idx.astype(jnp.int32)
    iidx = item_idx.astype(jnp.int32)
    packed = _pack_tables((gmf_user.T, gmf_item.T, mlp_user.T, mlp_item.T))
    gu, gi, mu, mi = _make_sc_gather()(uidx, iidx, *packed)
    Wpg = Wp[:D]
    Wph = Wp[D:]
    return _run_mlp(uidx, iidx, gu, gi, mu, mi, W1, b1, W2, b2, W3, b3,
                    Wpg, Wph, bp)


# speedup vs baseline: 2.5985x; 1.0924x over previous
import functools

import jax
import jax.numpy as jnp
from jax import lax
from jax.experimental import pallas as pl
from jax.experimental.pallas import tpu as pltpu
from jax.experimental.pallas import tpu_sc as plsc

B = 16384
D = 32
V = 1000000
LB = 2048
NBLK = (V + LB - 1) // LB
QR = LB // 4
HR = QR // 2
VP = NBLK * HR
NC, NS = 2, 16
NW = NC * NS
BPW = B // NW


def _pack_body(a_ref, b_ref, c_ref, d_ref, ao_ref, bo_ref, co_ref, do_ref):
    rows = lax.broadcasted_iota(jnp.int32, (128, D), 0)
    cols = lax.broadcasted_iota(jnp.int32, (128, D), 1)
    eqs = [(rows == cols + D * q).astype(jnp.float32) for q in range(4)]
    cdim = (((1,), (0,)), ((), ()))
    limit = V - pl.program_id(0) * LB
    lane = lax.broadcasted_iota(jnp.int32, (D, LB), 1)
    ok = lane < limit
    for src, dst in ((a_ref, ao_ref), (b_ref, bo_ref), (c_ref, co_ref),
                     (d_ref, do_ref)):
        x = jnp.where(ok, src[...], 0.0)
        acc = lax.dot_general(eqs[0], x[:, 0:QR], cdim,
                              preferred_element_type=jnp.float32)
        for q in range(1, 4):
            acc = acc + lax.dot_general(
                eqs[q], x[:, q * QR:(q + 1) * QR], cdim,
                preferred_element_type=jnp.float32)
        accT = acc.T
        u0 = lax.bitcast_convert_type(
            accT[0:HR, :].astype(jnp.bfloat16), jnp.uint16
        ).astype(jnp.uint32)
        u1 = lax.bitcast_convert_type(
            accT[HR:QR, :].astype(jnp.bfloat16), jnp.uint16
        ).astype(jnp.uint32)
        dst[...] = lax.bitcast_convert_type(u0 | (u1 << 16), jnp.int32)


def _pack_tables(tTs):
    in_spec = pl.BlockSpec((D, LB), lambda i: (0, i))
    out_spec = pl.BlockSpec((HR, 128), lambda i: (i, 0))
    return pl.pallas_call(
        _pack_body,
        grid=(NBLK,),
        in_specs=[in_spec] * 4,
        out_specs=[out_spec] * 4,
        out_shape=[jax.ShapeDtypeStruct((VP, 128), jnp.int32)] * 4,
        compiler_params=pltpu.CompilerParams(
            fuse_transposed_lhs_in_matmul=True),
    )(*tTs)


@functools.cache
def _make_sc_gather():
    mesh = plsc.VectorSubcoreMesh(core_axis_name="c", subcore_axis_name="s")

    @functools.partial(
        pl.kernel,
        out_type=[jax.ShapeDtypeStruct((B, 128), jnp.int32)] * 4,
        mesh=mesh,
        scratch_types=[
            pltpu.VMEM((BPW,), jnp.int32),
            pltpu.VMEM((BPW,), jnp.int32),
            pltpu.VMEM((BPW // 2, 128), jnp.int32),
            pltpu.VMEM((BPW // 2, 128), jnp.int32),
            pltpu.SemaphoreType.DMA,
            pltpu.SemaphoreType.DMA,
            pltpu.SemaphoreType.DMA,
        ],
    )
    def sc_gather(uidx_hbm, iidx_hbm, gu_hbm, gi_hbm, mu_hbm, mi_hbm,
                  gu_out, gi_out, mu_out, mi_out,
                  ju_v, ji_v, rows_a, rows_b, sema, semb, semo):
        wid = lax.axis_index("s") * NC + lax.axis_index("c")
        base = wid * BPW
        pltpu.sync_copy(uidx_hbm.at[pl.ds(base, BPW)], ju_v)
        pltpu.sync_copy(iidx_hbm.at[pl.ds(base, BPW)], ji_v)

        def to_packed(k, _):
            s = pl.ds(k * 16, 16)
            ju = ju_v[s]
            ji = ji_v[s]
            ju_v[s] = ((ju >> 11) << 8) | (ju & (HR - 1))
            ji_v[s] = ((ji >> 11) << 8) | (ji & (HR - 1))
            return ()

        lax.fori_loop(0, BPW // 16, to_packed, (), unroll=4)

        CH = BPW // 2
        bufs = (rows_a, rows_b)
        sems = (sema, semb)
        rounds = []
        for tbl, jv, out in ((gu_hbm, ju_v, gu_out), (gi_hbm, ji_v, gi_out),
                             (mu_hbm, ju_v, mu_out), (mi_hbm, ji_v, mi_out)):
            for c in range(2):
                rounds.append((tbl, jv, out, c))
        n = len(rounds)
        gathers = [None] * n
        stores = [None] * n

        def finish(i):
            _, _, out, c = rounds[i]
            gathers[i].wait()
            stores[i] = pltpu.async_copy(
                bufs[i % 2], out.at[pl.ds(base + c * CH, CH)], semo)

        for i, (tbl, jv, out, c) in enumerate(rounds):
            if i >= 2:
                stores[i - 2].wait()
            gathers[i] = pltpu.async_copy(
                tbl.at[jv.at[pl.ds(c * CH, CH)]], bufs[i % 2], sems[i % 2])
            if i >= 1:
                finish(i - 1)
        finish(n - 1)
        stores[n - 2].wait()
        stores[n - 1].wait()

    return sc_gather


BLK = 2048


def _mlp_body(uidx_ref, iidx_ref, gu_ref, gi_ref, mu_ref, mi_ref, w1_ref,
              b1_ref, w2_ref, b2_ref, w3_ref, b3_ref, wpg_ref, wph_ref,
              bp_ref, out_ref):
    uidx = uidx_ref[...]
    iidx = iidx_ref[...]
    uq = (uidx >> 9) & 3
    iq = (iidx >> 9) & 3
    us = jnp.broadcast_to(((uidx >> 8) & 1) * 16, (BLK, D))
    ish = jnp.broadcast_to(((iidx >> 8) & 1) * 16, (BLK, D))

    def pick(raw, q, shift):
        acc = jnp.zeros((BLK, D), jnp.int32)
        for c in range(4):
            sel = jnp.broadcast_to(q == c, (BLK, D))
            acc = acc + jnp.where(sel, raw[:, c * D:(c + 1) * D], 0)
        bits = (acc >> shift).astype(jnp.uint16)
        return lax.bitcast_convert_type(bits, jnp.bfloat16).astype(
            jnp.float32)

    gmf = pick(gu_ref[...], uq, us) * pick(gi_ref[...], iq, ish)
    x = jnp.concatenate([pick(mu_ref[...], uq, us),
                         pick(mi_ref[...], iq, ish)], axis=1)
    h = jnp.maximum(jnp.dot(x, w1_ref[...],
                            preferred_element_type=jnp.float32) + b1_ref[...],
                    0.0)
    h = jnp.maximum(jnp.dot(h, w2_ref[...],
                            preferred_element_type=jnp.float32) + b2_ref[...],
                    0.0)
    h = jnp.maximum(jnp.dot(h, w3_ref[...],
                            preferred_element_type=jnp.float32) + b3_ref[...],
                    0.0)
    logit = (jnp.dot(gmf, wpg_ref[...], preferred_element_type=jnp.float32)
             + jnp.dot(h, wph_ref[...], preferred_element_type=jnp.float32)
             + bp_ref[0, 0])
    out_ref[...] = jax.nn.sigmoid(logit)


def _run_mlp(uidx, iidx, gu, gi, mu, mi, W1, b1, W2, b2, W3, b3, Wpg, Wph,
             bp):
    grid = (B // BLK,)
    raw_spec = pl.BlockSpec((BLK, 128), lambda i: (i, 0))
    idx_spec = pl.BlockSpec((BLK, 1), lambda i: (i, 0))

    def whole(shape):
        return pl.BlockSpec(shape, lambda i: (0,) * len(shape))

    out = pl.pallas_call(
        _mlp_body,
        grid=grid,
        in_specs=[
            idx_spec, idx_spec,
            raw_spec, raw_spec, raw_spec, raw_spec,
            whole((64, 32)), whole((1, 32)),
            whole((32, 16)), whole((1, 16)),
            whole((16, 8)), whole((1, 8)),
            whole((32, 1)), whole((8, 1)), whole((1, 1)),
        ],
        out_specs=pl.BlockSpec((BLK, 1), lambda i: (i, 0)),
        out_shape=jax.ShapeDtypeStruct((B, 1), jnp.float32),
    )(uidx.reshape(B, 1), iidx.reshape(B, 1), gu, gi, mu, mi,
      W1, b1.reshape(1, 32), W2,
      b2.reshape(1, 16), W3, b3.reshape(1, 8), Wpg, Wph, bp.reshape(1, 1))
    return out.reshape(B)


def kernel(user_idx, item_idx, gmf_user, gmf_item, mlp_user, mlp_item,
           W1, b1, W2, b2, W3, b3, Wp, bp):
    uidx = user_idx.astype(jnp.int32)
    iidx = item_idx.astype(jnp.int32)
    packed = _pack_tables((gmf_user.T, gmf_item.T, mlp_user.T, mlp_item.T))
    gu, gi, mu, mi = _make_sc_gather()(uidx, iidx, *packed)
    Wpg = Wp[:D]
    Wph = Wp[D:]
    return _run_mlp(uidx, iidx, gu, gi, mu, mi, W1, b1, W2, b2, W3, b3,
                    Wpg, Wph, bp)
